# baseline (speedup 1.0000x reference)
import functools

import jax
import jax.numpy as jnp
from jax.experimental import pallas as pl
from jax.experimental.pallas import tpu as pltpu

VOXEL = jnp.array([0.8, 0.8, 0.8], dtype=jnp.float32)
PC_MIN = jnp.array([-51.2, -51.2, -3.2], dtype=jnp.float32)
PC_MAX = jnp.array([51.2, 51.2, 3.2], dtype=jnp.float32)
GRID = (128, 128, 8)
NUM_VOX = GRID[0] * GRID[1] * GRID[2]
C = 32


def _mlp_kernel(feats_ref, w_ref, b_ref, out_ref):
    f = feats_ref[...]
    w = w_ref[...]
    out_ref[...] = jax.nn.relu(
        jax.lax.dot_general(f, w, (((1,), (0,)), ((), ())),
                            preferred_element_type=jnp.float32)
        + b_ref[...][None, :])


def _mlp(feats, W, b):
    M = feats.shape[0]
    BLK = 4096
    return pl.pallas_call(
        _mlp_kernel,
        grid=(M // BLK,),
        in_specs=[
            pl.BlockSpec((BLK, 16), lambda i: (i, 0)),
            pl.BlockSpec((16, C), lambda i: (0, 0)),
            pl.BlockSpec((C,), lambda i: (0,)),
        ],
        out_specs=pl.BlockSpec((BLK, C), lambda i: (i, 0)),
        out_shape=jax.ShapeDtypeStruct((M, C), jnp.float32),
    )(feats, W, b)


def _process_cloud(points, W16, b):
    coords = jnp.floor((points - PC_MIN) / VOXEL).astype(jnp.int32)
    coords = jnp.clip(coords, 0, jnp.array(GRID, dtype=jnp.int32) - 1)
    vid = (coords[:, 0] * GRID[1] + coords[:, 1]) * GRID[2] + coords[:, 2]
    cnt = jax.ops.segment_sum(jnp.ones((points.shape[0],), jnp.float32), vid,
                              num_segments=NUM_VOX)
    denom = jnp.maximum(cnt, 1.0)
    sum_pts = jax.ops.segment_sum(points, vid, num_segments=NUM_VOX)
    mean_pts = sum_pts / denom[:, None]
    f_cluster = points - mean_pts[vid]
    centers = PC_MIN + (coords.astype(jnp.float32) + 0.5) * VOXEL
    f_center = points - centers
    feats = jnp.concatenate(
        [points, f_cluster, f_center,
         jnp.zeros((points.shape[0], 7), jnp.float32)], axis=-1)
    pf = _mlp(feats, W16, b)
    vf = jax.ops.segment_sum(pf, vid, num_segments=NUM_VOX) / denom[:, None]
    return vf, pf


def kernel(pc0s, pc1s, W, b, training_flag):
    B, N, _ = pc0s.shape
    W16 = jnp.concatenate([W, jnp.zeros((7, C), W.dtype)], axis=0)
    proc = jax.vmap(lambda p: _process_cloud(p, W16, b))
    vf0, pf0 = proc(pc0s)
    vf1, _ = proc(pc1s)
    all_voxel_feats = jnp.stack([vf0, vf1], axis=0)
    ts = jnp.full((B,), 1000.0, dtype=jnp.float32)
    nkey = jax.random.key(42)
    pc0_noise = jax.random.normal(jax.random.fold_in(nkey, 0), (B, 4 * N, 3),
                                  dtype=jnp.float32)
    pc1_noise = jax.random.normal(jax.random.fold_in(nkey, 1), (B, 4 * N, 3),
                                  dtype=jnp.float32)
    return (all_voxel_feats, vf0, pf0, pc0_noise, pc1_noise, ts)


# speedup vs baseline: 2.5304x; 2.5304x over previous
import functools

import jax
import jax.numpy as jnp
from jax import lax
from jax.experimental import pallas as pl
from jax.experimental.pallas import tpu as pltpu
from jax.experimental.pallas import tpu_sc as plsc

GRID = (128, 128, 8)
NUM_VOX = GRID[0] * GRID[1] * GRID[2]
C = 32

NCLOUD = 4
N = 65536
NC, NS = 2, 16
PPT = N // NS
CHUNK = 128
NCH = PPT // CHUNK
ACC_ROWS = 2 * NUM_VOX


def _k1_body(ptsT_ref, zc_ref,
             gT_ref, gc_ref, vid_ref,
             shx, shy, shz, shc,
             ptv, idx_v, gx, gy, gz, gcv, ones_v,
             sem_l, sem_s, sem_g):
    core = lax.axis_index("c")
    sub = lax.axis_index("s")
    tb = sub * PPT

    one16 = jnp.ones((16,), jnp.float32)
    for i in range(CHUNK // 16):
        ones_v[pl.ds(i * 16, 16)] = one16

    stripe = ACC_ROWS // NS
    zrow = sub * stripe
    hs = [pltpu.async_copy(zc_ref, s.at[pl.ds(zrow, stripe)], sem_l)
          for s in (shx, shy, shz, shc)]
    for h in hs:
        h.wait()
    plsc.subcore_barrier()

    for lc in range(2):
        g = 2 * core + lc
        hs = [pltpu.async_copy(ptsT_ref.at[pl.ds((g * 3 + i) * N + tb, PPT)],
                               ptv.at[pl.ds(i * PPT, PPT)], sem_l)
              for i in range(3)]
        for h in hs:
            h.wait()

        def vid_loop(j, _):
            for gi in range(CHUNK // 16):
                off = j * CHUNK + gi * 16
                x = ptv[pl.ds(off, 16)]
                y = ptv[pl.ds(PPT + off, 16)]
                z = ptv[pl.ds(2 * PPT + off, 16)]
                cx = ((x - (-51.2)) / 0.8).astype(jnp.int32)
                cy = ((y - (-51.2)) / 0.8).astype(jnp.int32)
                cz = ((z - (-3.2)) / 0.8).astype(jnp.int32)
                cx = jnp.minimum(jnp.maximum(cx, 0), GRID[0] - 1)
                cy = jnp.minimum(jnp.maximum(cy, 0), GRID[1] - 1)
                cz = jnp.minimum(jnp.maximum(cz, 0), GRID[2] - 1)
                vid = (cx * GRID[1] + cy) * GRID[2] + cz + lc * NUM_VOX
                idx_v[lc * NCH + j, 0, pl.ds(gi * 16, 16)] = vid
            return 0

        lax.fori_loop(0, NCH, vid_loop, 0)

        def sc_loop(j0, _):
            hs2 = []
            for jj in range(8):
                j = j0 * 8 + jj
                irow = idx_v.at[lc * NCH + j, 0]
                hs2.append(pltpu.async_copy(
                    ptv.at[pl.ds(j * CHUNK, CHUNK)], shx.at[irow],
                    sem_s, add=True))
                hs2.append(pltpu.async_copy(
                    ptv.at[pl.ds(PPT + j * CHUNK, CHUNK)], shy.at[irow],
                    sem_s, add=True))
                hs2.append(pltpu.async_copy(
                    ptv.at[pl.ds(2 * PPT + j * CHUNK, CHUNK)], shz.at[irow],
                    sem_s, add=True))
                hs2.append(pltpu.async_copy(ones_v, shc.at[irow],
                                            sem_s, add=True))
            for h in hs2:
                h.wait()
            return 0

        lax.fori_loop(0, NCH // 8, sc_loop, 0)

    plsc.subcore_barrier()

    for lc in range(2):
        g = 2 * core + lc

        def ga_loop(j0, _):
            hs2 = []
            for jj in range(8):
                j = j0 * 8 + jj
                irow = idx_v.at[lc * NCH + j, 0]
                dst = pl.ds(j * CHUNK, CHUNK)
                hs2.append(pltpu.async_copy(shx.at[irow], gx.at[dst], sem_g))
                hs2.append(pltpu.async_copy(shy.at[irow], gy.at[dst], sem_g))
                hs2.append(pltpu.async_copy(shz.at[irow], gz.at[dst], sem_g))
                hs2.append(pltpu.async_copy(shc.at[irow], gcv.at[dst], sem_g))
            for h in hs2:
                h.wait()
            return 0

        lax.fori_loop(0, NCH // 8, ga_loop, 0)

        hs = [
            pltpu.async_copy(gx, gT_ref.at[pl.ds((g * 3 + 0) * N + tb, PPT)], sem_l),
            pltpu.async_copy(gy, gT_ref.at[pl.ds((g * 3 + 1) * N + tb, PPT)], sem_l),
            pltpu.async_copy(gz, gT_ref.at[pl.ds((g * 3 + 2) * N + tb, PPT)], sem_l),
            pltpu.async_copy(gcv, gc_ref.at[pl.ds(g * N + tb, PPT)], sem_l),
            pltpu.async_copy(
                idx_v.at[pl.ds(lc * NCH, NCH)],
                vid_ref.at[pl.ds(g * (N // CHUNK) + tb // CHUNK, NCH)],
                sem_l),
        ]
        for h in hs:
            h.wait()


def _k1(ptsT, zc):
    mesh = plsc.VectorSubcoreMesh(core_axis_name="c", subcore_axis_name="s")
    f = pl.kernel(
        _k1_body,
        out_type=(
            jax.ShapeDtypeStruct((NCLOUD * 3 * N,), jnp.float32),
            jax.ShapeDtypeStruct((NCLOUD * N,), jnp.float32),
            jax.ShapeDtypeStruct((NCLOUD * N // CHUNK, 1, CHUNK), jnp.int32),
        ),
        mesh=mesh,
        scratch_types=(
            pltpu.VMEM_SHARED((ACC_ROWS,), jnp.float32),
            pltpu.VMEM_SHARED((ACC_ROWS,), jnp.float32),
            pltpu.VMEM_SHARED((ACC_ROWS,), jnp.float32),
            pltpu.VMEM_SHARED((ACC_ROWS,), jnp.float32),
            pltpu.VMEM((3 * PPT,), jnp.float32),
            pltpu.VMEM((2 * NCH, 1, CHUNK), jnp.int32),
            pltpu.VMEM((PPT,), jnp.float32),
            pltpu.VMEM((PPT,), jnp.float32),
            pltpu.VMEM((PPT,), jnp.float32),
            pltpu.VMEM((PPT,), jnp.float32),
            pltpu.VMEM((CHUNK,), jnp.float32),
            pltpu.SemaphoreType.DMA,
            pltpu.SemaphoreType.DMA,
            pltpu.SemaphoreType.DMA,
        ),
    )
    return f(ptsT, zc)


def _mlp_kernel(feats_ref, w_ref, b_ref, dinv_ref, pf_ref, ps_ref):
    f = feats_ref[0]
    w = w_ref[...]
    pf = jax.nn.relu(
        lax.dot_general(w, f, (((0,), (0,)), ((), ())),
                        preferred_element_type=jnp.float32)
        + b_ref[0][:, None])
    pf_ref[0] = pf
    ps_ref[0] = pf * dinv_ref[0]


def _mlp(featsT, W, b2, dinv3):
    BLK = 8192
    grid = (NCLOUD, N // BLK)
    return pl.pallas_call(
        _mlp_kernel,
        grid=grid,
        in_specs=[
            pl.BlockSpec((1, 9, BLK), lambda g, i: (g, 0, i)),
            pl.BlockSpec((9, C), lambda g, i: (0, 0)),
            pl.BlockSpec((1, C), lambda g, i: (0, 0)),
            pl.BlockSpec((1, 1, BLK), lambda g, i: (g, 0, i)),
        ],
        out_specs=[
            pl.BlockSpec((1, C, BLK), lambda g, i: (g, 0, i)),
            pl.BlockSpec((1, C, BLK), lambda g, i: (g, 0, i)),
        ],
        out_shape=[
            jax.ShapeDtypeStruct((NCLOUD, C, N), jnp.float32),
            jax.ShapeDtypeStruct((NCLOUD, C, N), jnp.float32),
        ],
    )(featsT, W, b2, dinv3)


def kernel(pc0s, pc1s, W, b, training_flag):
    B, n, _ = pc0s.shape
    pts = jnp.concatenate([pc0s, pc1s], axis=0)
    ptsT = jnp.transpose(pts, (0, 2, 1))
    zc = jnp.zeros((ACC_ROWS // NS,), jnp.float32)
    gTf, gcf, vidc = _k1(ptsT.reshape(-1), zc)
    gT = gTf.reshape(NCLOUD, 3, N)
    gc = gcf.reshape(NCLOUD, N)
    off = (jnp.arange(NCLOUD, dtype=jnp.int32) % 2) * NUM_VOX
    vid = vidc.reshape(NCLOUD, N) - off[:, None]

    pc_minT = jnp.array([-51.2, -51.2, -3.2], jnp.float32).reshape(1, 3, 1)
    voxelT = jnp.array([0.8, 0.8, 0.8], jnp.float32).reshape(1, 3, 1)
    denom = jnp.maximum(gc, 1.0)
    meanT = gT / denom[:, None, :]
    f_clusterT = ptsT - meanT
    coordsT = jnp.floor((ptsT - pc_minT) / voxelT).astype(jnp.int32)
    gmaxT = jnp.array(GRID, jnp.int32).reshape(1, 3, 1) - 1
    coordsT = jnp.clip(coordsT, 0, gmaxT)
    centersT = pc_minT + (coordsT.astype(jnp.float32) + 0.5) * voxelT
    f_centerT = ptsT - centersT
    featsT = jnp.concatenate([ptsT, f_clusterT, f_centerT], axis=1)

    dinv3 = (1.0 / denom).reshape(NCLOUD, 1, N)
    pfT, psT = _mlp(featsT, W, b.reshape(1, C), dinv3)

    pf_scaled = jnp.transpose(psT, (0, 2, 1))
    vf = jax.vmap(
        lambda p_, v_: jax.ops.segment_sum(p_, v_, num_segments=NUM_VOX)
    )(pf_scaled, vid)
    all_voxel_feats = vf.reshape(2, B, NUM_VOX, C)
    vf0 = all_voxel_feats[0]
    pf0 = jnp.transpose(pfT[:B], (0, 2, 1))

    ts = jnp.full((B,), 1000.0, dtype=jnp.float32)
    nkey = jax.random.key(42)
    pc0_noise = jax.random.normal(jax.random.fold_in(nkey, 0), (B, 4 * n, 3),
                                  dtype=jnp.float32)
    pc1_noise = jax.random.normal(jax.random.fold_in(nkey, 1), (B, 4 * n, 3),
                                  dtype=jnp.float32)
    return (all_voxel_feats, vf0, pf0, pc0_noise, pc1_noise, ts)


# speedup vs baseline: 6.9317x; 2.7394x over previous
import functools

import jax
import jax.numpy as jnp
from jax import lax
from jax.experimental import pallas as pl
from jax.experimental.pallas import tpu as pltpu
from jax.experimental.pallas import tpu_sc as plsc

GRID = (128, 128, 8)
NUM_VOX = GRID[0] * GRID[1] * GRID[2]
C = 32

NCLOUD = 4
N = 65536
NC, NS = 2, 16
PPT = N // NS
CHUNK = 128
NCH = PPT // CHUNK
VSTRIPE = NUM_VOX // NS


def _k1_body(ptsT_ref, zc_ref,
             gT_ref, gc_ref, vid_ref,
             shx0, shy0, shz0, shc0, shx1, shy1, shz1, shc1,
             ptv, idx_v, gx, gy, gz, gcv, ones_v,
             sem_l, sem_s, sem_g):
    core = lax.axis_index("c")
    sub = lax.axis_index("s")
    tb = sub * PPT
    bufs = ((shx0, shy0, shz0, shc0), (shx1, shy1, shz1, shc1))

    one16 = jnp.ones((16,), jnp.float32)
    for i in range(CHUNK // 16):
        ones_v[pl.ds(i * 16, 16)] = one16

    zrow = sub * VSTRIPE
    hs = [pltpu.async_copy(zc_ref, s.at[pl.ds(zrow, VSTRIPE)], sem_l)
          for bl in bufs for s in bl]
    for h in hs:
        h.wait()
    plsc.subcore_barrier()

    for lc in range(2):
        g = 2 * core + lc
        shx, shy, shz, shc = bufs[lc]
        hs = [pltpu.async_copy(ptsT_ref.at[pl.ds((g * 3 + i) * N + tb, PPT)],
                               ptv.at[pl.ds(i * PPT, PPT)], sem_l)
              for i in range(3)]
        for h in hs:
            h.wait()

        def vid_loop(j, _):
            for gi in range(CHUNK // 16):
                off = j * CHUNK + gi * 16
                x = ptv[pl.ds(off, 16)]
                y = ptv[pl.ds(PPT + off, 16)]
                z = ptv[pl.ds(2 * PPT + off, 16)]
                cx = ((x - (-51.2)) / 0.8).astype(jnp.int32)
                cy = ((y - (-51.2)) / 0.8).astype(jnp.int32)
                cz = ((z - (-3.2)) / 0.8).astype(jnp.int32)
                cx = jnp.minimum(jnp.maximum(cx, 0), GRID[0] - 1)
                cy = jnp.minimum(jnp.maximum(cy, 0), GRID[1] - 1)
                cz = jnp.minimum(jnp.maximum(cz, 0), GRID[2] - 1)
                vid = (cx * GRID[1] + cy) * GRID[2] + cz
                idx_v[lc * NCH + j, 0, pl.ds(gi * 16, 16)] = vid
            return 0

        lax.fori_loop(0, NCH, vid_loop, 0)

        def sc_loop(j0, _):
            hs2 = []
            for jj in range(8):
                j = j0 * 8 + jj
                irow = idx_v.at[lc * NCH + j, 0]
                hs2.append(pltpu.async_copy(
                    ptv.at[pl.ds(j * CHUNK, CHUNK)], shx.at[irow],
                    sem_s, add=True))
                hs2.append(pltpu.async_copy(
                    ptv.at[pl.ds(PPT + j * CHUNK, CHUNK)], shy.at[irow],
                    sem_s, add=True))
                hs2.append(pltpu.async_copy(
                    ptv.at[pl.ds(2 * PPT + j * CHUNK, CHUNK)], shz.at[irow],
                    sem_s, add=True))
                hs2.append(pltpu.async_copy(ones_v, shc.at[irow],
                                            sem_s, add=True))
            for h in hs2:
                h.wait()
            return 0

        lax.fori_loop(0, NCH // 8, sc_loop, 0)

    plsc.subcore_barrier()

    for lc in range(2):
        g = 2 * core + lc
        shx, shy, shz, shc = bufs[lc]

        def ga_loop(j0, _):
            hs2 = []
            for jj in range(8):
                j = j0 * 8 + jj
                irow = idx_v.at[lc * NCH + j, 0]
                dst = pl.ds(j * CHUNK, CHUNK)
                hs2.append(pltpu.async_copy(shx.at[irow], gx.at[dst], sem_g))
                hs2.append(pltpu.async_copy(shy.at[irow], gy.at[dst], sem_g))
                hs2.append(pltpu.async_copy(shz.at[irow], gz.at[dst], sem_g))
                hs2.append(pltpu.async_copy(shc.at[irow], gcv.at[dst], sem_g))
            for h in hs2:
                h.wait()
            return 0

        lax.fori_loop(0, NCH // 8, ga_loop, 0)

        hs = [
            pltpu.async_copy(gx, gT_ref.at[pl.ds((g * 3 + 0) * N + tb, PPT)], sem_l),
            pltpu.async_copy(gy, gT_ref.at[pl.ds((g * 3 + 1) * N + tb, PPT)], sem_l),
            pltpu.async_copy(gz, gT_ref.at[pl.ds((g * 3 + 2) * N + tb, PPT)], sem_l),
            pltpu.async_copy(gcv, gc_ref.at[pl.ds(g * N + tb, PPT)], sem_l),
            pltpu.async_copy(
                idx_v.at[pl.ds(lc * NCH, NCH)],
                vid_ref.at[pl.ds(g * (N // CHUNK) + tb // CHUNK, NCH)],
                sem_l),
        ]
        for h in hs:
            h.wait()


def _k1(ptsT, zc):
    mesh = plsc.VectorSubcoreMesh(core_axis_name="c", subcore_axis_name="s")
    f = pl.kernel(
        _k1_body,
        out_type=(
            jax.ShapeDtypeStruct((NCLOUD * 3 * N,), jnp.float32),
            jax.ShapeDtypeStruct((NCLOUD * N,), jnp.float32),
            jax.ShapeDtypeStruct((NCLOUD * N // CHUNK, 1, CHUNK), jnp.int32),
        ),
        mesh=mesh,
        scratch_types=(
            *[pltpu.VMEM_SHARED((NUM_VOX,), jnp.float32) for _ in range(8)],
            pltpu.VMEM((3 * PPT,), jnp.float32),
            pltpu.VMEM((2 * NCH, 1, CHUNK), jnp.int32),
            pltpu.VMEM((PPT,), jnp.float32),
            pltpu.VMEM((PPT,), jnp.float32),
            pltpu.VMEM((PPT,), jnp.float32),
            pltpu.VMEM((PPT,), jnp.float32),
            pltpu.VMEM((CHUNK,), jnp.float32),
            pltpu.SemaphoreType.DMA,
            pltpu.SemaphoreType.DMA,
            pltpu.SemaphoreType.DMA,
        ),
    )
    return f(ptsT, zc)


def _k3_body(psT_ref, vid_ref, zc_ref,
             avfT_ref,
             b0, b1, b2, b3, b4, b5, b6, b7,
             colv, vidv,
             sem_l, sem_s):
    core = lax.axis_index("c")
    sub = lax.axis_index("s")
    tb = sub * PPT
    bufs = (b0, b1, b2, b3, b4, b5, b6, b7)
    zrow = sub * VSTRIPE

    for q in range(2):
        def phase(g, _):
            cq = 2 * core + q
            hs = [pltpu.async_copy(zc_ref, s.at[pl.ds(zrow, VSTRIPE)], sem_l)
                  for s in bufs]
            for h in hs:
                h.wait()
            plsc.subcore_barrier()
            hs = [pltpu.async_copy(
                vid_ref.at[pl.ds(g * (N // CHUNK) + tb // CHUNK, NCH)],
                vidv, sem_l)]
            for c in range(8):
                hs.append(pltpu.async_copy(
                    psT_ref.at[pl.ds(((g * 4 + cq) * 8 + c) * N + tb, PPT)],
                    colv.at[pl.ds(c * PPT, PPT)], sem_l))
            for h in hs:
                h.wait()

            def sc_loop(j0, _):
                hs2 = []
                for jj in range(8):
                    j = j0 * 8 + jj
                    irow = vidv.at[j, 0]
                    for c in range(8):
                        hs2.append(pltpu.async_copy(
                            colv.at[pl.ds(c * PPT + j * CHUNK, CHUNK)],
                            bufs[c].at[irow], sem_s, add=True))
                for h in hs2:
                    h.wait()
                return 0

            lax.fori_loop(0, NCH // 8, sc_loop, 0)
            plsc.subcore_barrier()

            hs = []
            for c in range(8):
                hs.append(pltpu.async_copy(
                    bufs[c].at[pl.ds(zrow, VSTRIPE)],
                    avfT_ref.at[pl.ds(((g * 4 + cq) * 8 + c) * NUM_VOX + zrow,
                                      VSTRIPE)],
                    sem_l))
            for h in hs:
                h.wait()
            plsc.subcore_barrier()
            return 0

        lax.fori_loop(0, NCLOUD, phase, 0)


def _k3(psT, vidc, zc):
    mesh = plsc.VectorSubcoreMesh(core_axis_name="c", subcore_axis_name="s")
    f = pl.kernel(
        _k3_body,
        out_type=(
            jax.ShapeDtypeStruct((NCLOUD * C * NUM_VOX,), jnp.float32),
        ),
        mesh=mesh,
        scratch_types=(
            *[pltpu.VMEM_SHARED((NUM_VOX,), jnp.float32) for _ in range(8)],
            pltpu.VMEM((8 * PPT,), jnp.float32),
            pltpu.VMEM((NCH, 1, CHUNK), jnp.int32),
            pltpu.SemaphoreType.DMA,
            pltpu.SemaphoreType.DMA,
        ),
    )
    return f(psT, vidc, zc)[0]


def _mlp_kernel(feats_ref, w_ref, b_ref, dinv_ref, pf_ref, ps_ref):
    f = feats_ref[0]
    w = w_ref[...]
    pf = jax.nn.relu(
        lax.dot_general(w, f, (((0,), (0,)), ((), ())),
                        preferred_element_type=jnp.float32)
        + b_ref[0][:, None])
    pf_ref[0] = pf
    ps_ref[0] = pf * dinv_ref[0]


def _mlp(featsT, W, b2, dinv3):
    BLK = 8192
    grid = (NCLOUD, N // BLK)
    return pl.pallas_call(
        _mlp_kernel,
        grid=grid,
        in_specs=[
            pl.BlockSpec((1, 9, BLK), lambda g, i: (g, 0, i)),
            pl.BlockSpec((9, C), lambda g, i: (0, 0)),
            pl.BlockSpec((1, C), lambda g, i: (0, 0)),
            pl.BlockSpec((1, 1, BLK), lambda g, i: (g, 0, i)),
        ],
        out_specs=[
            pl.BlockSpec((1, C, BLK), lambda g, i: (g, 0, i)),
            pl.BlockSpec((1, C, BLK), lambda g, i: (g, 0, i)),
        ],
        out_shape=[
            jax.ShapeDtypeStruct((NCLOUD, C, N), jnp.float32),
            jax.ShapeDtypeStruct((NCLOUD, C, N), jnp.float32),
        ],
    )(featsT, W, b2, dinv3)


def kernel(pc0s, pc1s, W, b, training_flag):
    B, n, _ = pc0s.shape
    pts = jnp.concatenate([pc0s, pc1s], axis=0)
    ptsT = jnp.transpose(pts, (0, 2, 1))
    zc = jnp.zeros((VSTRIPE,), jnp.float32)
    gTf, gcf, vidc = _k1(ptsT.reshape(-1), zc)
    gT = gTf.reshape(NCLOUD, 3, N)
    gc = gcf.reshape(NCLOUD, N)

    pc_minT = jnp.array([-51.2, -51.2, -3.2], jnp.float32).reshape(1, 3, 1)
    voxelT = jnp.array([0.8, 0.8, 0.8], jnp.float32).reshape(1, 3, 1)
    denom = jnp.maximum(gc, 1.0)
    meanT = gT / denom[:, None, :]
    f_clusterT = ptsT - meanT
    coordsT = jnp.floor((ptsT - pc_minT) / voxelT).astype(jnp.int32)
    gmaxT = jnp.array(GRID, jnp.int32).reshape(1, 3, 1) - 1
    coordsT = jnp.clip(coordsT, 0, gmaxT)
    centersT = pc_minT + (coordsT.astype(jnp.float32) + 0.5) * voxelT
    f_centerT = ptsT - centersT
    featsT = jnp.concatenate([ptsT, f_clusterT, f_centerT], axis=1)

    dinv3 = (1.0 / denom).reshape(NCLOUD, 1, N)
    pfT, psT = _mlp(featsT, W, b.reshape(1, C), dinv3)

    avfT = _k3(psT.reshape(-1), vidc, zc)
    all_voxel_feats = jnp.transpose(
        avfT.reshape(2, B, C, NUM_VOX), (0, 1, 3, 2))
    vf0 = all_voxel_feats[0]
    pf0 = jnp.transpose(pfT[:B], (0, 2, 1))

    ts = jnp.full((B,), 1000.0, dtype=jnp.float32)
    nkey = jax.random.key(42)
    pc0_noise = jax.random.normal(jax.random.fold_in(nkey, 0), (B, 4 * n, 3),
                                  dtype=jnp.float32)
    pc1_noise = jax.random.normal(jax.random.fold_in(nkey, 1), (B, 4 * n, 3),
                                  dtype=jnp.float32)
    return (all_voxel_feats, vf0, pf0, pc0_noise, pc1_noise, ts)
